# baseline (speedup 1.0000x reference)
import functools

import jax
import jax.numpy as jnp
from jax import lax
from jax.experimental import pallas as pl
from jax.experimental.pallas import tpu as pltpu
from jax.experimental.pallas import tpu_sc as plsc

HIST = 50
BATCH = 4096
D = 64


def _sc_embed(table, xt):
    info = plsc.get_sparse_core_info()
    nc, ns = info.num_cores, info.num_subcores
    nw = nc * ns
    bb = BATCH // nw
    dt = D // 8
    mesh = plsc.VectorSubcoreMesh(core_axis_name="c", subcore_axis_name="s")

    @functools.partial(
        pl.kernel,
        out_type=jax.ShapeDtypeStruct((HIST, dt, nw, 8, bb), jnp.float32),
        mesh=mesh,
        scratch_types=[
            [pltpu.VMEM((bb,), jnp.int32) for _ in range(2)],
            [pltpu.VMEM((bb, D), jnp.float32) for _ in range(2)],
            [pltpu.VMEM((dt, 8, bb), jnp.float32) for _ in range(2)],
            [pltpu.SemaphoreType.DMA for _ in range(2)],
            [pltpu.SemaphoreType.DMA for _ in range(2)],
        ],
        compiler_params=pltpu.CompilerParams(
            use_tc_tiling_on_sc=False, needs_layout_passes=False
        ),
    )
    def body(tab_hbm, xt_hbm, out_hbm, idxs, gbufs, tbufs, gsems, ssems):
        wid = lax.axis_index("s") * nc + lax.axis_index("c")
        b0 = wid * bb

        def load_idx(h, s):
            pltpu.sync_copy(xt_hbm.at[h, pl.ds(b0, bb)], idxs[s])

        def gather_desc(s):
            return pltpu.make_async_copy(tab_hbm.at[idxs[s]], gbufs[s], gsems[s])

        def store_desc(h, s):
            return pltpu.make_async_copy(tbufs[s], out_hbm.at[h, :, wid], ssems[s])

        def transpose(s):
            rows = lax.broadcasted_iota(jnp.int32, (16,), 0)
            for bg in range(0, bb, 16):
                ridx = rows + bg
                for dd in range(D):
                    vals = plsc.load_gather(
                        gbufs[s], [ridx, jnp.full((16,), dd, jnp.int32)]
                    )
                    tbufs[s][dd // 8, dd % 8, pl.ds(bg, 16)] = vals

        load_idx(0, 0)
        gather_desc(0).start()

        def group(g, carry):
            for p in range(2):
                h = 2 * g + p

                @pl.when(h + 1 < HIST)
                def _():
                    load_idx(h + 1, 1 - p)
                    gather_desc(1 - p).start()

                gather_desc(p).wait()

                @pl.when(h >= 2)
                def _():
                    store_desc(h - 2, p).wait()

                transpose(p)
                store_desc(h, p).start()
            return carry

        lax.fori_loop(0, HIST // 2, group, 0)
        store_desc(HIST - 2, 0).wait()
        store_desc(HIST - 1, 1).wait()

    return body(table, xt)


def kernel(x, table):
    xt = jnp.swapaxes(x, 0, 1).astype(jnp.int32)
    out_l = _sc_embed(table, xt)
    out = jnp.transpose(out_l, (2, 4, 0, 1, 3)).reshape(BATCH, HIST, D)
    return out


# speedup vs baseline: 1.4162x; 1.4162x over previous
import functools

import jax
import jax.numpy as jnp
from jax import lax
from jax.experimental import pallas as pl
from jax.experimental.pallas import tpu as pltpu
from jax.experimental.pallas import tpu_sc as plsc

HIST = 50
BATCH = 4096
D = 64


def _sc_embed(table, xt):
    info = plsc.get_sparse_core_info()
    nc, ns = info.num_cores, info.num_subcores
    nw = nc * ns
    bb = BATCH // nw
    dt = D // 8
    mesh = plsc.VectorSubcoreMesh(core_axis_name="c", subcore_axis_name="s")

    @functools.partial(
        pl.kernel,
        out_type=jax.ShapeDtypeStruct((HIST, dt, nw, 8, bb), jnp.float32),
        mesh=mesh,
        scratch_types=[
            [pltpu.VMEM((bb,), jnp.int32) for _ in range(2)],
            [pltpu.VMEM((bb, D), jnp.float32) for _ in range(2)],
            [pltpu.VMEM((dt, 8, bb), jnp.float32) for _ in range(2)],
            [pltpu.VMEM((16, 17), jnp.float32) for _ in range(4)],
            [pltpu.SemaphoreType.DMA for _ in range(2)],
            [pltpu.SemaphoreType.DMA for _ in range(2)],
        ],
        compiler_params=pltpu.CompilerParams(
            use_tc_tiling_on_sc=False, needs_layout_passes=False
        ),
    )
    def body(tab_hbm, xt_hbm, out_hbm, idxs, gbufs, tbufs, blks, gsems, ssems):
        wid = lax.axis_index("s") * nc + lax.axis_index("c")
        b0 = wid * bb

        def load_idx(h, s):
            pltpu.sync_copy(xt_hbm.at[h, pl.ds(b0, bb)], idxs[s])

        def gather_desc(s):
            return pltpu.make_async_copy(tab_hbm.at[idxs[s]], gbufs[s], gsems[s])

        def store_desc(h, s):
            return pltpu.make_async_copy(tbufs[s], out_hbm.at[h, :, wid], ssems[s])

        def transpose(s):
            rows = lax.broadcasted_iota(jnp.int32, (16,), 0)

            def bg_body(bg, carry):
                b16 = bg * 16
                for dg in range(D // 16):
                    blk = blks[dg]
                    for i in range(16):
                        blk[i, pl.ds(0, 16)] = gbufs[s][
                            b16 + i, pl.ds(dg * 16, 16)
                        ]
                for dg in range(D // 16):
                    blk = blks[dg]
                    for j in range(16):
                        col = plsc.load_gather(
                            blk, [rows, jnp.full((16,), j, jnp.int32)]
                        )
                        dd = dg * 16 + j
                        tbufs[s][dd // 8, dd % 8, pl.ds(b16, 16)] = col
                return carry

            lax.fori_loop(0, bb // 16, bg_body, 0)

        load_idx(0, 0)
        gather_desc(0).start()

        def group(g, carry):
            for p in range(2):
                h = 2 * g + p

                @pl.when(h + 1 < HIST)
                def _():
                    load_idx(h + 1, 1 - p)
                    gather_desc(1 - p).start()

                gather_desc(p).wait()

                @pl.when(h >= 2)
                def _():
                    store_desc(h - 2, p).wait()

                transpose(p)
                store_desc(h, p).start()
            return carry

        lax.fori_loop(0, HIST // 2, group, 0)
        store_desc(HIST - 2, 0).wait()
        store_desc(HIST - 1, 1).wait()

    return body(table, xt)


def kernel(x, table):
    xt = jnp.swapaxes(x, 0, 1).astype(jnp.int32)
    out_l = _sc_embed(table, xt)
    out = jnp.transpose(out_l, (2, 4, 0, 1, 3)).reshape(BATCH, HIST, D)
    return out


# speedup vs baseline: 1.6321x; 1.1525x over previous
import functools

import jax
import jax.numpy as jnp
from jax import lax
from jax.experimental import pallas as pl
from jax.experimental.pallas import tpu as pltpu
from jax.experimental.pallas import tpu_sc as plsc

HIST = 50
BATCH = 4096
D = 64


def _sc_embed(table, xt):
    info = plsc.get_sparse_core_info()
    nc, ns = info.num_cores, info.num_subcores
    nw = nc * ns
    bb = BATCH // nw
    dt = D // 8
    mesh = plsc.VectorSubcoreMesh(core_axis_name="c", subcore_axis_name="s")

    @functools.partial(
        pl.kernel,
        out_type=jax.ShapeDtypeStruct((HIST, dt, nw, 8, bb), jnp.float32),
        mesh=mesh,
        scratch_types=[
            [pltpu.VMEM((bb,), jnp.int32) for _ in range(2)],
            [pltpu.VMEM((bb, D), jnp.float32) for _ in range(2)],
            [pltpu.VMEM((dt, 8, bb), jnp.float32) for _ in range(2)],
            [pltpu.VMEM((16, 17), jnp.float32) for _ in range(4)],
            [pltpu.SemaphoreType.DMA for _ in range(2)],
            [pltpu.SemaphoreType.DMA for _ in range(2)],
        ],
        compiler_params=pltpu.CompilerParams(
            use_tc_tiling_on_sc=False, needs_layout_passes=False
        ),
    )
    def body(tab_hbm, xt_hbm, out_hbm, idxs, gbufs, tbufs, blks, gsems, ssems):
        wid = lax.axis_index("s") * nc + lax.axis_index("c")
        b0 = wid * bb

        def load_idx(h, s):
            pltpu.sync_copy(xt_hbm.at[h, pl.ds(b0, bb)], idxs[s])

        def gather_desc(s):
            return pltpu.make_async_copy(tab_hbm.at[idxs[s]], gbufs[s], gsems[s])

        def store_desc(h, s):
            return pltpu.make_async_copy(tbufs[s], out_hbm.at[h, :, wid], ssems[s])

        def transpose(s):
            rows = lax.broadcasted_iota(jnp.int32, (16,), 0)

            def bg_body(bg, carry):
                b16 = bg * 16
                for dg in range(D // 16):
                    blk = blks[dg]
                    for i in range(16):
                        val = gbufs[s][b16 + i, pl.ds(dg * 16, 16)]
                        plsc.store_scatter(
                            blk, [rows, jnp.full((16,), i, jnp.int32)], val
                        )
                for dg in range(D // 16):
                    blk = blks[dg]
                    for j in range(16):
                        dd = dg * 16 + j
                        tbufs[s][dd // 8, dd % 8, pl.ds(b16, 16)] = blk[
                            j, pl.ds(0, 16)
                        ]
                return carry

            lax.fori_loop(0, bb // 16, bg_body, 0)

        load_idx(0, 0)
        gather_desc(0).start()

        def group(g, carry):
            for p in range(2):
                h = 2 * g + p

                @pl.when(h + 1 < HIST)
                def _():
                    load_idx(h + 1, 1 - p)
                    gather_desc(1 - p).start()

                gather_desc(p).wait()

                @pl.when(h >= 2)
                def _():
                    store_desc(h - 2, p).wait()

                transpose(p)
                store_desc(h, p).start()
            return carry

        lax.fori_loop(0, HIST // 2, group, 0)
        store_desc(HIST - 2, 0).wait()
        store_desc(HIST - 1, 1).wait()

    return body(table, xt)


def kernel(x, table):
    xt = jnp.swapaxes(x, 0, 1).astype(jnp.int32)
    out_l = _sc_embed(table, xt)
    out = jnp.transpose(out_l, (2, 4, 0, 1, 3)).reshape(BATCH, HIST, D)
    return out
